# baseline (speedup 1.0000x reference)
import functools

import jax
import jax.numpy as jnp
from jax.experimental import pallas as pl
from jax.experimental.pallas import tpu as pltpu


def _params(sem):
    return pltpu.CompilerParams(dimension_semantics=sem,
                                vmem_limit_bytes=60 * 1024 * 1024)


def _lat_kernel(x_ref, w_ref, b_ref, o_ref):
    o_ref[...] = (jnp.dot(x_ref[...], w_ref[...],
                          preferred_element_type=jnp.float32)
                  + b_ref[...]).astype(o_ref.dtype)


def _lateral(x, wm, b2):
    M, K = x.shape
    C = wm.shape[1]
    tm = min(2048, M // 2 if M >= 2048 else M)
    return pl.pallas_call(
        _lat_kernel,
        out_shape=jax.ShapeDtypeStruct((M, C), jnp.bfloat16),
        grid=(pl.cdiv(M, tm),),
        in_specs=[
            pl.BlockSpec((tm, K), lambda m: (m, 0)),
            pl.BlockSpec((K, C), lambda m: (0, 0)),
            pl.BlockSpec((1, C), lambda m: (0, 0)),
        ],
        out_specs=pl.BlockSpec((tm, C), lambda m: (m, 0)),
        compiler_params=_params(("parallel",)),
        cost_estimate=pl.CostEstimate(
            flops=2 * M * K * C, transcendentals=0,
            bytes_accessed=(M * K + K * C + M * C) * 2),
    )(x, wm, b2)


def _upadd_kernel(d_ref, f_ref, o_ref):
    d = d_ref[...]
    d2 = jnp.concatenate([d, d], axis=-1)
    o_ref[...] = d2[:, None, :, :] + f_ref[...]


def _upsample_add(deep, fine):
    N, Hs, Ws, C = deep.shape
    deep_f = deep.reshape(N * Hs, Ws, C)
    fine_f = fine.reshape(N * Hs, 2, Ws, 2 * C)
    nd = N * Hs
    tr = max(1, min(16, nd // 2))
    out = pl.pallas_call(
        _upadd_kernel,
        out_shape=jax.ShapeDtypeStruct(fine_f.shape, jnp.bfloat16),
        grid=(pl.cdiv(nd, tr),),
        in_specs=[
            pl.BlockSpec((tr, Ws, C), lambda i: (i, 0, 0)),
            pl.BlockSpec((tr, 2, Ws, 2 * C), lambda i: (i, 0, 0, 0)),
        ],
        out_specs=pl.BlockSpec((tr, 2, Ws, 2 * C), lambda i: (i, 0, 0, 0)),
        compiler_params=_params(("parallel",)),
    )(deep_f, fine_f)
    return out.reshape(fine.shape)


def _conv3_kernel(a_ref, h_ref, w_ref, b_ref, o_ref, rows_ref, im_ref):
    th = a_ref.shape[1]
    wp = o_ref.shape[2]
    c = a_ref.shape[3]
    rows_ref[0:th] = a_ref[0]
    rows_ref[th:th + 2] = h_ref[0]
    for dh in range(3):
        for dw in range(3):
            t = dh * 3 + dw
            im_ref[:, t * c:(t + 1) * c] = (
                rows_ref[dh:dh + th, dw:dw + wp, :].reshape(th * wp, c))
    o = (jnp.dot(im_ref[...], w_ref[...], preferred_element_type=jnp.float32)
         + b_ref[...])
    o_ref[...] = o.reshape(1, th, wp, o_ref.shape[3]).astype(o_ref.dtype)


def _conv3(x, w9, b2):
    N, H, W, C = x.shape
    th = min(32, H)
    xp = jnp.pad(x, ((0, 0), (1, 1), (1, 1), (0, 0)))
    out = pl.pallas_call(
        _conv3_kernel,
        out_shape=jax.ShapeDtypeStruct((N, H, W, C), jnp.bfloat16),
        grid=(N, H // th),
        in_specs=[
            pl.BlockSpec((1, th, W + 2, C), lambda n, i: (n, i, 0, 0)),
            pl.BlockSpec((1, 2, W + 2, C),
                         lambda n, i: (n, (i + 1) * th // 2, 0, 0)),
            pl.BlockSpec((9 * C, C), lambda n, i: (0, 0)),
            pl.BlockSpec((1, C), lambda n, i: (0, 0)),
        ],
        out_specs=pl.BlockSpec((1, th, W, C), lambda n, i: (n, i, 0, 0)),
        scratch_shapes=[
            pltpu.VMEM((th + 2, W + 2, C), jnp.bfloat16),
            pltpu.VMEM((th * W, 9 * C), jnp.bfloat16),
        ],
        compiler_params=_params(("parallel", "parallel")),
        cost_estimate=pl.CostEstimate(
            flops=2 * N * H * W * 9 * C * C, transcendentals=0,
            bytes_accessed=(N * (H + 2) * (W + 2) * C + 9 * C * C
                            + N * H * W * C) * 2),
    )(xp, xp, w9, b2)
    return out


def kernel(feat0, feat1, feat2,
           w1_0, w1_1, w1_2,
           b1_0, b1_1, b1_2,
           w3_0, w3_1, w3_2, w3_3, w3_4,
           b3_0, b3_1, b3_2, b3_3, b3_4):
    cd = jnp.bfloat16
    feats = [jnp.transpose(f, (0, 2, 3, 1)).astype(cd)
             for f in (feat0, feat1, feat2)]

    C = w1_0.shape[0]
    w1m = [jnp.transpose(w.reshape(w.shape[0], w.shape[1]), (1, 0)).astype(cd)
           for w in (w1_0, w1_1, w1_2)]
    b1p = [b.reshape(1, C).astype(jnp.float32) for b in (b1_0, b1_1, b1_2)]
    w3m = [jnp.transpose(w, (2, 3, 1, 0)).reshape(9 * C, C).astype(cd)
           for w in (w3_0, w3_1, w3_2, w3_3, w3_4)]
    b3p = [b.reshape(1, C).astype(jnp.float32)
           for b in (b3_0, b3_1, b3_2, b3_3, b3_4)]

    lats = []
    for f, wm, bp in zip(feats, w1m, b1p):
        N, H, W, Cin = f.shape
        y = _lateral(f.reshape(N * H * W, Cin), wm, bp)
        lats.append(y.reshape(N, H, W, C))

    sum4 = _upsample_add(lats[2], lats[1])
    sum3 = _upsample_add(sum4, lats[0])

    out3 = _conv3(sum3, w3m[0], b3p[0])
    out4 = _conv3(sum4, w3m[1], b3p[1])
    out5 = _conv3(lats[2], w3m[2], b3p[2])
    out6 = _conv3(out5, w3m[3], b3p[3])[:, ::2, ::2, :]
    out7 = _conv3(out6, w3m[4], b3p[4])[:, ::2, ::2, :]

    return [jnp.transpose(o, (0, 3, 1, 2)).astype(jnp.float32)
            for o in (out3, out4, out5, out6, out7)]
